# speedup vs baseline: 5.1328x; 5.1328x over previous
import functools

import jax
import jax.numpy as jnp
from jax import lax
from jax.experimental import pallas as pl
from jax.experimental.pallas import tpu as pltpu
from jax.experimental.pallas import tpu_sc as plsc

VOCAB = 100000
EMBD = 128
HIDDEN = 128

_INFO = plsc.get_sparse_core_info()
_NC, _NS = _INFO.num_cores, _INFO.num_subcores
_NW = _NC * _NS

_CHUNK = 128


def _proj_kernel(wv_ref, w_ref, out_ref):
    out_ref[...] = lax.dot_general(
        wv_ref[...], w_ref[...],
        dimension_numbers=(((1,), (1,)), ((), ())),
        preferred_element_type=jnp.float32,
    )


def _project_table(word_vectors, W_proj):
    rows_per_block = 1000
    grid = VOCAB // rows_per_block
    return pl.pallas_call(
        _proj_kernel,
        grid=(grid,),
        in_specs=[
            pl.BlockSpec((rows_per_block, EMBD), lambda i: (i, 0)),
            pl.BlockSpec((HIDDEN, EMBD), lambda i: (0, 0)),
        ],
        out_specs=pl.BlockSpec((rows_per_block, HIDDEN), lambda i: (i, 0)),
        out_shape=jax.ShapeDtypeStruct((VOCAB, HIDDEN), jnp.float32),
    )(word_vectors, W_proj)


def _make_gather(total, d):
    assert total % (_NW * _CHUNK) == 0
    per_worker = total // _NW
    n_chunks = per_worker // _CHUNK
    mesh = plsc.VectorSubcoreMesh(core_axis_name="c", subcore_axis_name="s")

    @functools.partial(
        pl.kernel,
        mesh=mesh,
        out_type=jax.ShapeDtypeStruct((total, d), jnp.float32),
        scratch_types=[
            pltpu.VMEM((_CHUNK,), jnp.int32),
            pltpu.VMEM((_CHUNK, d), jnp.float32),
            pltpu.SemaphoreType.DMA,
        ],
    )
    def gather(table_hbm, idx_hbm, out_hbm, idx_v, rows_v, sem):
        wid = lax.axis_index("s") * _NC + lax.axis_index("c")
        base = wid * per_worker

        @pl.loop(0, n_chunks)
        def _chunk(i):
            off = base + i * _CHUNK
            pltpu.sync_copy(idx_hbm.at[pl.ds(off, _CHUNK)], idx_v)
            pltpu.async_copy(table_hbm.at[idx_v], rows_v, sem).wait()
            pltpu.sync_copy(rows_v, out_hbm.at[pl.ds(off, _CHUNK)])

    return gather


@jax.jit
def kernel(x, word_vectors, W_proj):
    b, s = x.shape
    proj_table = _project_table(word_vectors, W_proj)
    flat_idx = x.reshape(-1).astype(jnp.int32)
    out = _make_gather(b * s, HIDDEN)(proj_table, flat_idx)
    return out.reshape(b, s, HIDDEN)


# speedup vs baseline: 8.3123x; 1.6194x over previous
import functools

import jax
import jax.numpy as jnp
from jax import lax
from jax.experimental import pallas as pl
from jax.experimental.pallas import tpu as pltpu
from jax.experimental.pallas import tpu_sc as plsc

VOCAB = 100000
EMBD = 128
HIDDEN = 128

_INFO = plsc.get_sparse_core_info()
_NC, _NS = _INFO.num_cores, _INFO.num_subcores
_NW = _NC * _NS

_CHUNK = 128


def _proj_kernel(wv_ref, w_ref, out_ref):
    out_ref[...] = lax.dot_general(
        wv_ref[...], w_ref[...],
        dimension_numbers=(((1,), (1,)), ((), ())),
        preferred_element_type=jnp.float32,
    )


def _project_table(word_vectors, W_proj):
    rows_per_block = 1000
    grid = VOCAB // rows_per_block
    return pl.pallas_call(
        _proj_kernel,
        grid=(grid,),
        in_specs=[
            pl.BlockSpec((rows_per_block, EMBD), lambda i: (i, 0)),
            pl.BlockSpec((HIDDEN, EMBD), lambda i: (0, 0)),
        ],
        out_specs=pl.BlockSpec((rows_per_block, HIDDEN), lambda i: (i, 0)),
        out_shape=jax.ShapeDtypeStruct((VOCAB, HIDDEN), jnp.float32),
    )(word_vectors, W_proj)


_K = 2
_NBUF = 2
_ROWS = _K * _CHUNK


def _make_gather(total, d):
    assert total % (_NW * _ROWS) == 0
    per_worker = total // _NW
    n_chunks = per_worker // _CHUNK
    n_groups = n_chunks // _K
    assert n_groups % _NBUF == 0 and n_groups >= 2 * _NBUF
    mesh = plsc.VectorSubcoreMesh(core_axis_name="c", subcore_axis_name="s")

    @functools.partial(
        pl.kernel,
        mesh=mesh,
        out_type=jax.ShapeDtypeStruct((total, d), jnp.float32),
        scratch_types=[
            pltpu.VMEM((n_chunks, _CHUNK), jnp.int32),
            pltpu.VMEM((_ROWS, d), jnp.float32),
            pltpu.VMEM((_ROWS, d), jnp.float32),
            pltpu.SemaphoreType.DMA,
            pltpu.SemaphoreType.DMA,
        ],
    )
    def gather(table_hbm, idx2d_hbm, out_hbm, idx_all, buf0, buf1, g0, g1):
        wid = lax.axis_index("s") * _NC + lax.axis_index("c")
        base = wid * per_worker
        bufs = (buf0, buf1)
        sems = (g0, g1)

        pltpu.sync_copy(idx2d_hbm.at[pl.ds(wid * n_chunks, n_chunks)], idx_all)

        def fire(grp, slot):
            for j in range(_K):
                pltpu.async_copy(
                    table_hbm.at[idx_all.at[grp * _K + j]],
                    bufs[slot].at[pl.ds(j * _CHUNK, _CHUNK)],
                    sems[slot],
                )

        def drain(slot):
            pltpu.make_async_copy(
                out_hbm.at[pl.ds(0, _ROWS)], bufs[slot], sems[slot]
            ).wait()

        for p in range(_NBUF):
            fire(p, p)

        @pl.loop(0, n_groups, step=_NBUF)
        def _group(g):
            for p in range(_NBUF):
                grp = g + p
                drain(p)
                pltpu.sync_copy(bufs[p], out_hbm.at[pl.ds(base + grp * _ROWS, _ROWS)])

                @pl.when(grp + _NBUF < n_groups)
                def _refill():
                    fire(grp + _NBUF, p)

    return gather


@jax.jit
def kernel(x, word_vectors, W_proj):
    b, s = x.shape
    proj_table = _project_table(word_vectors, W_proj)
    flat_idx = x.reshape(-1).astype(jnp.int32)
    idx2d = flat_idx.reshape(-1, _CHUNK)
    out = _make_gather(b * s, HIDDEN)(proj_table, idx2d)
    return out.reshape(b, s, HIDDEN)
